# baseline (speedup 1.0000x reference)
import jax
import jax.numpy as jnp
from jax.experimental import pallas as pl
from jax.experimental.pallas import tpu as pltpu

S = 2048
HIDDEN = 1024
INTER = 4096
E = 8
TOPK = 2

LANES = 128
T = 1024
FT = 1024
NF = INTER // FT
R = S * TOPK
NT = R // T + (E - 1)


def _router_body(x_ref, gw_ref, logits_ref, w01_ref, i01_ref):
    x = x_ref[...]
    gw = gw_ref[...]
    logits = jnp.dot(x, gw, preferred_element_type=jnp.float32)
    logits_ref[...] = logits
    lane = jax.lax.broadcasted_iota(jnp.int32, logits.shape, 1)
    neg = jnp.float32(-1e30)
    ml = jnp.where(lane < E, logits, neg)
    m0 = jnp.max(ml, axis=1, keepdims=True)
    i0 = jnp.min(jnp.where(ml == m0, lane, E), axis=1, keepdims=True)
    ml2 = jnp.where(lane == i0, neg, ml)
    m1 = jnp.max(ml2, axis=1, keepdims=True)
    i1 = jnp.min(jnp.where(ml2 == m1, lane, E), axis=1, keepdims=True)
    b = jnp.exp(m1 - m0)
    w0 = 1.0 / (1.0 + b)
    w1 = b / (1.0 + b)
    w01_ref[...] = jnp.where(lane == 0, w0, jnp.where(lane == 1, w1, 0.0))
    i01_ref[...] = jnp.where(lane == 0, i0, jnp.where(lane == 1, i1, 0))


def _router(x, gate_w_pad):
    return pl.pallas_call(
        _router_body,
        out_shape=(
            jax.ShapeDtypeStruct((S, LANES), jnp.float32),
            jax.ShapeDtypeStruct((S, LANES), jnp.float32),
            jax.ShapeDtypeStruct((S, LANES), jnp.int32),
        ),
    )(x, gate_w_pad)


def _moe_body(te_ref, tv_ref, x_ref, w_ref, wg_ref, wv_ref, wo_ref, y_ref):
    f = pl.program_id(1)

    @pl.when(tv_ref[pl.program_id(0)] != 0)
    def _():
        x = x_ref[...].astype(jnp.bfloat16)
        g = jnp.dot(x, wg_ref[0].astype(jnp.bfloat16),
                    preferred_element_type=jnp.float32)
        v = jnp.dot(x, wv_ref[0].astype(jnp.bfloat16),
                    preferred_element_type=jnp.float32)
        h = jax.nn.gelu(g, approximate=True) * v
        part = jnp.dot(h.astype(jnp.bfloat16), wo_ref[0].astype(jnp.bfloat16),
                       preferred_element_type=jnp.float32)
        prev = jnp.where(f == 0, 0.0, y_ref[...])
        acc = prev + part
        y_ref[...] = jnp.where(f == NF - 1, acc * w_ref[:, 0:1], acc)


def _grouped_mlp(tile_e, tile_valid, x_slots, w_slots, wg, wv, wo):
    grid_spec = pltpu.PrefetchScalarGridSpec(
        num_scalar_prefetch=2,
        grid=(NT, NF),
        in_specs=[
            pl.BlockSpec((T, HIDDEN), lambda t, f, te, tv: (t, 0)),
            pl.BlockSpec((T, E), lambda t, f, te, tv: (t, 0)),
            pl.BlockSpec((1, HIDDEN, FT), lambda t, f, te, tv: (te[t], 0, f)),
            pl.BlockSpec((1, HIDDEN, FT), lambda t, f, te, tv: (te[t], 0, f)),
            pl.BlockSpec((1, FT, HIDDEN), lambda t, f, te, tv: (te[t], f, 0)),
        ],
        out_specs=pl.BlockSpec((T, HIDDEN), lambda t, f, te, tv: (t, 0)),
    )
    return pl.pallas_call(
        _moe_body,
        grid_spec=grid_spec,
        out_shape=jax.ShapeDtypeStruct((NT * T, HIDDEN), jnp.float32),
    )(tile_e, tile_valid, x_slots, w_slots, wg, wv, wo)


@jax.jit
def kernel(hidden_states, gate_w, wg, wv, wo):
    x = hidden_states[0]

    gate_w_pad = jnp.zeros((HIDDEN, LANES), jnp.float32).at[:, :E].set(gate_w)
    logits128, w01, i01 = _router(x, gate_w_pad)
    router_logits = logits128[:, :E]
    sel = i01[:, :TOPK]
    rw = w01[:, :TOPK]

    e_flat = jnp.reshape(sel, (R,))
    w_flat = jnp.reshape(rw, (R,))
    t_flat = jnp.arange(R, dtype=jnp.int32) // TOPK
    onehot = (e_flat[:, None] == jnp.arange(E, dtype=jnp.int32)[None, :]).astype(jnp.int32)
    csum = jnp.cumsum(onehot, axis=0)
    rank = jnp.take_along_axis(csum, e_flat[:, None], axis=1)[:, 0] - 1
    counts = csum[-1]
    tiles_per = (counts + T - 1) // T
    cum_tiles = jnp.cumsum(tiles_per)
    tile_start = cum_tiles - tiles_per
    used = cum_tiles[-1]

    dest = (tile_start[e_flat] * T + rank).astype(jnp.int32)
    slot_token = jnp.zeros((NT * T,), jnp.int32).at[dest].set(t_flat)
    slot_w8 = jnp.zeros((NT * T, E), jnp.float32).at[dest].set(
        jnp.broadcast_to(w_flat[:, None], (R, E)))

    tile_ids = jnp.arange(NT, dtype=jnp.int32)
    tile_e_raw = jnp.sum((cum_tiles[None, :] <= tile_ids[:, None]).astype(jnp.int32), axis=1)
    last_e = jnp.sum((cum_tiles <= used - 1).astype(jnp.int32))
    tile_valid = (tile_ids < used).astype(jnp.int32)
    tile_e = jnp.where(tile_valid == 1, jnp.minimum(tile_e_raw, E - 1), last_e)

    x_slots = jnp.take(x, slot_token, axis=0)
    y_slots = _grouped_mlp(tile_e, tile_valid, x_slots, slot_w8, wg, wv, wo)

    s01 = jnp.reshape(dest, (S, TOPK))
    final = jnp.sum(jnp.take(y_slots, s01, axis=0), axis=1)

    return (final[None], router_logits[None])


# speedup vs baseline: 1.1468x; 1.1468x over previous
import jax
import jax.numpy as jnp
from jax.experimental import pallas as pl
from jax.experimental.pallas import tpu as pltpu

S = 2048
HIDDEN = 1024
INTER = 4096
E = 8
TOPK = 2

LANES = 128
T = 512
FT = 1024
NF = INTER // FT
R = S * TOPK
NT = R // T + (E - 1)


def _router_body(x_ref, gw_ref, logits_ref, w01_ref, i01_ref):
    x = x_ref[...]
    gw = gw_ref[...]
    logits = jnp.dot(x, gw, preferred_element_type=jnp.float32)
    logits_ref[...] = logits
    lane = jax.lax.broadcasted_iota(jnp.int32, logits.shape, 1)
    neg = jnp.float32(-1e30)
    ml = jnp.where(lane < E, logits, neg)
    m0 = jnp.max(ml, axis=1, keepdims=True)
    i0 = jnp.min(jnp.where(ml == m0, lane, E), axis=1, keepdims=True)
    ml2 = jnp.where(lane == i0, neg, ml)
    m1 = jnp.max(ml2, axis=1, keepdims=True)
    i1 = jnp.min(jnp.where(ml2 == m1, lane, E), axis=1, keepdims=True)
    b = jnp.exp(m1 - m0)
    w0 = 1.0 / (1.0 + b)
    w1 = b / (1.0 + b)
    w01_ref[...] = jnp.where(lane == 0, w0, jnp.where(lane == 1, w1, 0.0))
    i01_ref[...] = jnp.where(lane == 0, i0, jnp.where(lane == 1, i1, 0))


def _router(x, gate_w_pad):
    return pl.pallas_call(
        _router_body,
        out_shape=(
            jax.ShapeDtypeStruct((S, LANES), jnp.float32),
            jax.ShapeDtypeStruct((S, LANES), jnp.float32),
            jax.ShapeDtypeStruct((S, LANES), jnp.int32),
        ),
    )(x, gate_w_pad)


def _moe_body(te_ref, tv_ref, x_ref, w_ref, wg_ref, wv_ref, wo_ref, y_ref):
    f = pl.program_id(1)

    @pl.when(tv_ref[pl.program_id(0)] != 0)
    def _():
        x = x_ref[...]
        g = jnp.dot(x, wg_ref[0].astype(jnp.bfloat16),
                    preferred_element_type=jnp.float32)
        v = jnp.dot(x, wv_ref[0].astype(jnp.bfloat16),
                    preferred_element_type=jnp.float32)
        h = jax.nn.gelu(g, approximate=True) * v
        part = jnp.dot(h.astype(jnp.bfloat16), wo_ref[0].astype(jnp.bfloat16),
                       preferred_element_type=jnp.float32)
        prev = jnp.where(f == 0, 0.0, y_ref[...].astype(jnp.float32))
        acc = prev + part
        y_ref[...] = jnp.where(f == NF - 1, acc * w_ref[:, 0:1],
                               acc).astype(jnp.bfloat16)


def _grouped_mlp(tile_e, tile_valid, x_slots, w_slots, wg, wv, wo):
    grid_spec = pltpu.PrefetchScalarGridSpec(
        num_scalar_prefetch=2,
        grid=(NT, NF),
        in_specs=[
            pl.BlockSpec((T, HIDDEN), lambda t, f, te, tv: (t, 0)),
            pl.BlockSpec((T, E), lambda t, f, te, tv: (t, 0)),
            pl.BlockSpec((1, HIDDEN, FT), lambda t, f, te, tv: (te[t], 0, f)),
            pl.BlockSpec((1, HIDDEN, FT), lambda t, f, te, tv: (te[t], 0, f)),
            pl.BlockSpec((1, FT, HIDDEN), lambda t, f, te, tv: (te[t], f, 0)),
        ],
        out_specs=pl.BlockSpec((T, HIDDEN), lambda t, f, te, tv: (t, 0)),
    )
    return pl.pallas_call(
        _moe_body,
        grid_spec=grid_spec,
        out_shape=jax.ShapeDtypeStruct((NT * T, HIDDEN), jnp.bfloat16),
    )(tile_e, tile_valid, x_slots, w_slots, wg, wv, wo)


@jax.jit
def kernel(hidden_states, gate_w, wg, wv, wo):
    x = hidden_states[0]

    gate_w_pad = jnp.zeros((HIDDEN, LANES), jnp.float32).at[:, :E].set(gate_w)
    logits128, w01, i01 = _router(x, gate_w_pad)
    router_logits = logits128[:, :E]
    sel = i01[:, :TOPK]
    rw = w01[:, :TOPK]

    e_flat = jnp.reshape(sel, (R,))
    w_flat = jnp.reshape(rw, (R,))
    t_flat = jnp.arange(R, dtype=jnp.int32) // TOPK
    onehot = (e_flat[:, None] == jnp.arange(E, dtype=jnp.int32)[None, :]).astype(jnp.int32)
    csum = jnp.cumsum(onehot, axis=0)
    rank = jnp.take_along_axis(csum, e_flat[:, None], axis=1)[:, 0] - 1
    counts = csum[-1]
    tiles_per = (counts + T - 1) // T
    cum_tiles = jnp.cumsum(tiles_per)
    tile_start = cum_tiles - tiles_per
    used = cum_tiles[-1]

    dest = (tile_start[e_flat] * T + rank).astype(jnp.int32)
    slot_token = jnp.zeros((NT * T,), jnp.int32).at[dest].set(t_flat)
    slot_w8 = jnp.zeros((NT * T, E), jnp.float32).at[dest].set(
        jnp.broadcast_to(w_flat[:, None], (R, E)))

    tile_ids = jnp.arange(NT, dtype=jnp.int32)
    tile_e_raw = jnp.sum((cum_tiles[None, :] <= tile_ids[:, None]).astype(jnp.int32), axis=1)
    last_e = jnp.sum((cum_tiles <= used - 1).astype(jnp.int32))
    tile_valid = (tile_ids < used).astype(jnp.int32)
    tile_e = jnp.where(tile_valid == 1, jnp.minimum(tile_e_raw, E - 1), last_e)

    x_slots = jnp.take(x.astype(jnp.bfloat16), slot_token, axis=0)
    y_slots = _grouped_mlp(tile_e, tile_valid, x_slots, slot_w8, wg, wv, wo)

    s01 = jnp.reshape(dest, (S, TOPK))
    final = jnp.sum(jnp.take(y_slots, s01, axis=0).astype(jnp.float32), axis=1)

    return (final[None], router_logits[None])


# speedup vs baseline: 1.2127x; 1.0575x over previous
import jax
import jax.numpy as jnp
from jax.experimental import pallas as pl
from jax.experimental.pallas import tpu as pltpu

S = 2048
HIDDEN = 1024
INTER = 4096
E = 8
TOPK = 2

LANES = 128
T = 512
FT = 1024
NF = INTER // FT
R = S * TOPK
NT = R // T + (E - 1)


def _router_body(x_ref, gw_ref, logits_ref, w01_ref, i01_ref):
    x = x_ref[...]
    gw = gw_ref[...]
    logits = jnp.dot(x, gw, preferred_element_type=jnp.float32)
    logits_ref[...] = logits
    lane = jax.lax.broadcasted_iota(jnp.int32, logits.shape, 1)
    neg = jnp.float32(-1e30)
    ml = jnp.where(lane < E, logits, neg)
    m0 = jnp.max(ml, axis=1, keepdims=True)
    i0 = jnp.min(jnp.where(ml == m0, lane, E), axis=1, keepdims=True)
    ml2 = jnp.where(lane == i0, neg, ml)
    m1 = jnp.max(ml2, axis=1, keepdims=True)
    i1 = jnp.min(jnp.where(ml2 == m1, lane, E), axis=1, keepdims=True)
    b = jnp.exp(m1 - m0)
    w0 = 1.0 / (1.0 + b)
    w1 = b / (1.0 + b)
    w01_ref[...] = jnp.where(lane == 0, w0, jnp.where(lane == 1, w1, 0.0))
    i01_ref[...] = jnp.where(lane == 0, i0, jnp.where(lane == 1, i1, 0))


def _router(x, gate_w_pad):
    return pl.pallas_call(
        _router_body,
        out_shape=(
            jax.ShapeDtypeStruct((S, LANES), jnp.float32),
            jax.ShapeDtypeStruct((S, LANES), jnp.float32),
            jax.ShapeDtypeStruct((S, LANES), jnp.int32),
        ),
    )(x, gate_w_pad)


def _moe_body(te_ref, tv_ref, x_ref, wg_ref, wv_ref, wo_ref, y_ref):
    f = pl.program_id(1)

    @pl.when(tv_ref[pl.program_id(0)] != 0)
    def _():
        x = x_ref[...]
        g = jnp.dot(x, wg_ref[0].astype(jnp.bfloat16),
                    preferred_element_type=jnp.float32)
        v = jnp.dot(x, wv_ref[0].astype(jnp.bfloat16),
                    preferred_element_type=jnp.float32)
        h = jax.nn.gelu(g, approximate=True) * v
        part = jnp.dot(h.astype(jnp.bfloat16), wo_ref[0].astype(jnp.bfloat16),
                       preferred_element_type=jnp.float32)
        prev = jnp.where(f == 0, 0.0, y_ref[...].astype(jnp.float32))
        y_ref[...] = (prev + part).astype(jnp.bfloat16)


def _grouped_mlp(tile_e, tile_valid, x_slots, wg, wv, wo):
    grid_spec = pltpu.PrefetchScalarGridSpec(
        num_scalar_prefetch=2,
        grid=(NT, NF),
        in_specs=[
            pl.BlockSpec((T, HIDDEN), lambda t, f, te, tv: (t, 0)),
            pl.BlockSpec((1, HIDDEN, FT), lambda t, f, te, tv: (te[t], 0, f)),
            pl.BlockSpec((1, HIDDEN, FT), lambda t, f, te, tv: (te[t], 0, f)),
            pl.BlockSpec((1, FT, HIDDEN), lambda t, f, te, tv: (te[t], f, 0)),
        ],
        out_specs=pl.BlockSpec((T, HIDDEN), lambda t, f, te, tv: (t, 0)),
    )
    return pl.pallas_call(
        _moe_body,
        grid_spec=grid_spec,
        out_shape=jax.ShapeDtypeStruct((NT * T, HIDDEN), jnp.bfloat16),
    )(tile_e, tile_valid, x_slots, wg, wv, wo)


@jax.jit
def kernel(hidden_states, gate_w, wg, wv, wo):
    x = hidden_states[0]

    gate_w_pad = jnp.zeros((HIDDEN, LANES), jnp.float32).at[:, :E].set(gate_w)
    logits128, w01, i01 = _router(x, gate_w_pad)
    router_logits = logits128[:, :E]
    sel = i01[:, :TOPK]
    rw = w01[:, :TOPK]

    e_flat = jnp.reshape(sel, (R,))
    onehot = (e_flat[:, None] == jnp.arange(E, dtype=jnp.int32)[None, :]).astype(jnp.int32)
    csum = jnp.cumsum(onehot, axis=0)
    rank = jnp.take_along_axis(csum, e_flat[:, None], axis=1)[:, 0] - 1
    counts = csum[-1]
    tiles_per = (counts + T - 1) // T
    cum_tiles = jnp.cumsum(tiles_per)
    tile_start = cum_tiles - tiles_per
    used = cum_tiles[-1]

    dest = (tile_start[e_flat] * T + rank).astype(jnp.int32)

    tile_ids = jnp.arange(NT, dtype=jnp.int32)
    tile_e_raw = jnp.sum((cum_tiles[None, :] <= tile_ids[:, None]).astype(jnp.int32), axis=1)
    last_e = jnp.sum((cum_tiles <= used - 1).astype(jnp.int32))
    tile_valid = (tile_ids < used).astype(jnp.int32)
    tile_e = jnp.where(tile_valid == 1, jnp.minimum(tile_e_raw, E - 1), last_e)

    x2 = jnp.repeat(x.astype(jnp.bfloat16), TOPK, axis=0)
    x_slots = jnp.zeros((NT * T, HIDDEN), jnp.bfloat16).at[dest].set(x2)
    y_slots = _grouped_mlp(tile_e, tile_valid, x_slots, wg, wv, wo)

    s01 = jnp.reshape(dest, (S, TOPK))
    y01 = jnp.take(y_slots, s01, axis=0).astype(jnp.float32)
    final = jnp.sum(y01 * rw[:, :, None], axis=1)

    return (final[None], router_logits[None])
